# baseline (speedup 1.0000x reference)
import math

import jax
import jax.numpy as jnp
from jax.experimental import pallas as pl

K = 50
ROWS = 64


def _topk_kernel(cost_ref, vals_ref, eidx_ref):
    n = cost_ref.shape[1]
    blk = pl.program_id(0)
    v0 = cost_ref[...]
    lane_iota = jax.lax.broadcasted_iota(jnp.int32, (ROWS, n), 1)
    out_lane = jax.lax.broadcasted_iota(jnp.int32, (ROWS, K), 1)

    def body(i, carry):
        v, av, ai = carry
        m = jnp.min(v, axis=1, keepdims=True)
        midx = jnp.where(v == m, lane_iota, n)
        idx = jnp.min(midx, axis=1, keepdims=True)
        v = jnp.where(midx == idx, jnp.inf, v)
        av = jnp.where(out_lane == i, m, av)
        ai = jnp.where(out_lane == i, idx, ai)
        return v, av, ai

    av0 = jnp.zeros((ROWS, K), jnp.float32)
    ai0 = jnp.zeros((ROWS, K), jnp.int32)
    _, av, ai = jax.lax.fori_loop(0, K, body, (v0, av0, ai0))

    row = blk * ROWS + jax.lax.broadcasted_iota(jnp.int32, (ROWS, 1), 0)
    off = (row // n) * n
    vals_ref[...] = av
    eidx_ref[0] = jnp.broadcast_to(row, (ROWS, K))
    eidx_ref[1] = ai + off


def _edge_attr_kernel(v_ref, w_ref, b_ref, out_ref):
    out_ref[...] = v_ref[...] * w_ref[...] + b_ref[...]


def kernel(cost_matrix, init_embedding, W, b):
    B, n, _ = cost_matrix.shape
    R = B * n
    cost = cost_matrix.reshape(R, n)

    vals, eidx = pl.pallas_call(
        _topk_kernel,
        grid=(R // ROWS,),
        in_specs=[pl.BlockSpec((ROWS, n), lambda i: (i, 0))],
        out_specs=[
            pl.BlockSpec((ROWS, K), lambda i: (i, 0)),
            pl.BlockSpec((2, ROWS, K), lambda i: (0, i, 0)),
        ],
        out_shape=[
            jax.ShapeDtypeStruct((R, K), jnp.float32),
            jax.ShapeDtypeStruct((2, R, K), jnp.int32),
        ],
    )(cost)

    E = R * K
    EB = math.gcd(E, 8000)
    attr = pl.pallas_call(
        _edge_attr_kernel,
        grid=(E // EB,),
        in_specs=[
            pl.BlockSpec((EB, 1), lambda i: (i, 0)),
            pl.BlockSpec((1, 64), lambda i: (0, 0)),
            pl.BlockSpec((1, 64), lambda i: (0, 0)),
        ],
        out_specs=pl.BlockSpec((EB, 64), lambda i: (i, 0)),
        out_shape=jax.ShapeDtypeStruct((E, 64), jnp.float32),
    )(vals.reshape(E, 1), W.reshape(1, 64), b.reshape(1, 64))

    x = init_embedding.reshape(R, -1)
    edge_index = eidx.reshape(2, E)
    return x, edge_index, attr


# speedup vs baseline: 1.6042x; 1.6042x over previous
import math

import jax
import jax.numpy as jnp
from jax.experimental import pallas as pl

K = 50
LANES = 128


def _topk_kernel(cost_ref, vals_ref, eidx_ref):
    n = cost_ref.shape[0]
    blk = pl.program_id(0)
    v0 = cost_ref[...]
    sub_iota = jax.lax.broadcasted_iota(jnp.int32, (n, LANES), 0)
    out_sub = jax.lax.broadcasted_iota(jnp.int32, (K, LANES), 0)

    def body(i, carry):
        v, av, ai = carry
        m = jnp.min(v, axis=0, keepdims=True)
        midx = jnp.where(v == m, sub_iota, n)
        idx = jnp.min(midx, axis=0, keepdims=True)
        v = jnp.where(midx == idx, jnp.inf, v)
        av = jnp.where(out_sub == i, m, av)
        ai = jnp.where(out_sub == i, idx, ai)
        return v, av, ai

    av0 = jnp.zeros((K, LANES), jnp.float32)
    ai0 = jnp.zeros((K, LANES), jnp.int32)
    _, av, ai = jax.lax.fori_loop(0, K, body, (v0, av0, ai0))

    row = blk * LANES + jax.lax.broadcasted_iota(jnp.int32, (1, LANES), 1)
    off = (row // n) * n
    vals_ref[...] = av
    eidx_ref[0] = jnp.broadcast_to(row, (K, LANES))
    eidx_ref[1] = ai + off


def _edge_attr_kernel(v_ref, w_ref, b_ref, out_ref):
    out_ref[...] = v_ref[...] * w_ref[...] + b_ref[...]


def kernel(cost_matrix, init_embedding, W, b):
    B, n, _ = cost_matrix.shape
    R = B * n
    costT = jnp.transpose(cost_matrix, (2, 0, 1)).reshape(n, R)

    valsT, eidxT = pl.pallas_call(
        _topk_kernel,
        grid=(R // LANES,),
        in_specs=[pl.BlockSpec((n, LANES), lambda i: (0, i))],
        out_specs=[
            pl.BlockSpec((K, LANES), lambda i: (0, i)),
            pl.BlockSpec((2, K, LANES), lambda i: (0, 0, i)),
        ],
        out_shape=[
            jax.ShapeDtypeStruct((K, R), jnp.float32),
            jax.ShapeDtypeStruct((2, K, R), jnp.int32),
        ],
    )(costT)

    vals = valsT.T
    eidx = jnp.transpose(eidxT, (0, 2, 1))

    E = R * K
    EB = math.gcd(E, 8000)
    attr = pl.pallas_call(
        _edge_attr_kernel,
        grid=(E // EB,),
        in_specs=[
            pl.BlockSpec((EB, 1), lambda i: (i, 0)),
            pl.BlockSpec((1, 64), lambda i: (0, 0)),
            pl.BlockSpec((1, 64), lambda i: (0, 0)),
        ],
        out_specs=pl.BlockSpec((EB, 64), lambda i: (i, 0)),
        out_shape=jax.ShapeDtypeStruct((E, 64), jnp.float32),
    )(vals.reshape(E, 1), W.reshape(1, 64), b.reshape(1, 64))

    x = init_embedding.reshape(R, -1)
    edge_index = eidx.reshape(2, E)
    return x, edge_index, attr


# speedup vs baseline: 3.0829x; 1.9218x over previous
import math

import jax
import jax.numpy as jnp
from jax.experimental import pallas as pl

K = 50
LANES = 128
NV = 128


def _lex_lt(av, ai, bv, bi):
    return (av < bv) | ((av == bv) & (ai < bi))


def _bitonic_topk(V, I, sub8):
    def cx(i, j, desc):
        av, ai, bv, bi = V[i], I[i], V[j], I[j]
        swap = _lex_lt(bv, bi, av, ai)
        if desc is True:
            swap = ~swap
        elif desc is not False:
            swap = swap != desc
        V[i] = jnp.where(swap, bv, av)
        V[j] = jnp.where(swap, av, bv)
        I[i] = jnp.where(swap, bi, ai)
        I[j] = jnp.where(swap, ai, bi)

    for k in (2, 4, 8, 16, 32, 64):
        s = k // 2
        while s >= 1:
            for half, hdesc in ((0, False), (64, True)):
                for q in range(64):
                    if q & s:
                        continue
                    cx(half + q, half + (q | s), ((q & k) != 0) != hdesc)
            s //= 2

    for q in range(64):
        av, ai, bv, bi = V[q], I[q], V[64 + q], I[64 + q]
        bl = _lex_lt(bv, bi, av, ai)
        V[q] = jnp.where(bl, bv, av)
        I[q] = jnp.where(bl, bi, ai)
    V = V[:64]
    I = I[:64]

    def merge64(desc):
        for s in (32, 16, 8, 4, 2, 1):
            for q in range(64):
                if q & s:
                    continue
                cx(q, q | s, desc)

    def prune_sublane(partner_fn):
        for q in range(64):
            pv = partner_fn(V[q])
            pi = partner_fn(I[q])
            bl = _lex_lt(pv, pi, V[q], I[q])
            V[q] = jnp.where(bl, pv, V[q])
            I[q] = jnp.where(bl, pi, I[q])

    merge64((sub8 & 4) != 0)
    prune_sublane(lambda x: jnp.roll(x, 4, axis=0))
    merge64((sub8 & 2) != 0)
    up2 = (sub8 & 2) == 0
    prune_sublane(
        lambda x: jnp.where(up2, jnp.roll(x, -2, axis=0), jnp.roll(x, 2, axis=0)))
    merge64((sub8 & 1) != 0)
    up1 = (sub8 & 1) == 0
    prune_sublane(
        lambda x: jnp.where(up1, jnp.roll(x, -1, axis=0), jnp.roll(x, 1, axis=0)))
    merge64(False)
    return V, I


def _topk_kernel(cost_ref, vals_ref, eidx_ref):
    n = cost_ref.shape[0]
    blk = pl.program_id(0)
    sub8 = jax.lax.broadcasted_iota(jnp.int32, (8, LANES), 0)

    V = []
    I = []
    for j in range(n // 8):
        V.append(cost_ref[8 * j:8 * j + 8, :])
        I.append(sub8 + (8 * j))
    for j in range(n // 8, NV):
        V.append(jnp.full((8, LANES), jnp.inf, jnp.float32))
        I.append(sub8 + (8 * j))

    V, I = _bitonic_topk(V, I, sub8)

    row = blk * LANES + jax.lax.broadcasted_iota(jnp.int32, (1, LANES), 1)
    off = (row // n) * n
    eidx_ref[0] = jnp.broadcast_to(row, (K, LANES))
    for t in range(K // 8 + 1):
        lo = 8 * t
        hi = min(lo + 8, K)
        cv = V[lo]
        ci = I[lo]
        for u in range(1, hi - lo):
            cv = jnp.where(sub8 == u, V[lo + u], cv)
            ci = jnp.where(sub8 == u, I[lo + u], ci)
        vals_ref[lo:hi, :] = cv[: hi - lo, :]
        eidx_ref[1, lo:hi, :] = (ci + off)[: hi - lo, :]


def _edge_attr_kernel(v_ref, w_ref, b_ref, out_ref):
    out_ref[...] = v_ref[...] * w_ref[...] + b_ref[...]


def kernel(cost_matrix, init_embedding, W, b):
    B, n, _ = cost_matrix.shape
    R = B * n
    costT = jnp.transpose(cost_matrix, (2, 0, 1)).reshape(n, R)

    valsT, eidxT = pl.pallas_call(
        _topk_kernel,
        grid=(R // LANES,),
        in_specs=[pl.BlockSpec((n, LANES), lambda i: (0, i))],
        out_specs=[
            pl.BlockSpec((K, LANES), lambda i: (0, i)),
            pl.BlockSpec((2, K, LANES), lambda i: (0, 0, i)),
        ],
        out_shape=[
            jax.ShapeDtypeStruct((K, R), jnp.float32),
            jax.ShapeDtypeStruct((2, K, R), jnp.int32),
        ],
    )(costT)

    vals = valsT.T
    eidx = jnp.transpose(eidxT, (0, 2, 1))

    E = R * K
    EB = math.gcd(E, 8000)
    attr = pl.pallas_call(
        _edge_attr_kernel,
        grid=(E // EB,),
        in_specs=[
            pl.BlockSpec((EB, 1), lambda i: (i, 0)),
            pl.BlockSpec((1, 64), lambda i: (0, 0)),
            pl.BlockSpec((1, 64), lambda i: (0, 0)),
        ],
        out_specs=pl.BlockSpec((EB, 64), lambda i: (i, 0)),
        out_shape=jax.ShapeDtypeStruct((E, 64), jnp.float32),
    )(vals.reshape(E, 1), W.reshape(1, 64), b.reshape(1, 64))

    x = init_embedding.reshape(R, -1)
    edge_index = eidx.reshape(2, E)
    return x, edge_index, attr


# speedup vs baseline: 3.2007x; 1.0382x over previous
import math

import jax
import jax.numpy as jnp
from jax.experimental import pallas as pl

K = 50
LANES = 128
NV = 128


def _lex_lt(av, ai, bv, bi):
    return (av < bv) | ((av == bv) & (ai < bi))


def _bitonic_topk(V, I, sub8):
    def cx(i, j, desc):
        av, ai, bv, bi = V[i], I[i], V[j], I[j]
        swap = _lex_lt(bv, bi, av, ai)
        if desc is True:
            swap = ~swap
        elif desc is not False:
            swap = swap != desc
        V[i] = jnp.where(swap, bv, av)
        V[j] = jnp.where(swap, av, bv)
        I[i] = jnp.where(swap, bi, ai)
        I[j] = jnp.where(swap, ai, bi)

    for k in (2, 4, 8, 16, 32, 64):
        s = k // 2
        while s >= 1:
            for half, hdesc in ((0, False), (64, True)):
                for q in range(64):
                    if q & s:
                        continue
                    cx(half + q, half + (q | s), ((q & k) != 0) != hdesc)
            s //= 2

    for q in range(64):
        av, ai, bv, bi = V[q], I[q], V[64 + q], I[64 + q]
        bl = _lex_lt(bv, bi, av, ai)
        V[q] = jnp.where(bl, bv, av)
        I[q] = jnp.where(bl, bi, ai)
    V = V[:64]
    I = I[:64]

    def merge64(desc):
        for s in (32, 16, 8, 4, 2, 1):
            for q in range(64):
                if q & s:
                    continue
                cx(q, q | s, desc)

    def prune_sublane(partner_fn):
        for q in range(64):
            pv = partner_fn(V[q])
            pi = partner_fn(I[q])
            bl = _lex_lt(pv, pi, V[q], I[q])
            V[q] = jnp.where(bl, pv, V[q])
            I[q] = jnp.where(bl, pi, I[q])

    merge64((sub8 & 4) != 0)
    prune_sublane(lambda x: jnp.roll(x, 4, axis=0))
    merge64((sub8 & 2) != 0)
    up2 = (sub8 & 2) == 0
    prune_sublane(
        lambda x: jnp.where(up2, jnp.roll(x, -2, axis=0), jnp.roll(x, 2, axis=0)))
    merge64((sub8 & 1) != 0)
    up1 = (sub8 & 1) == 0
    prune_sublane(
        lambda x: jnp.where(up1, jnp.roll(x, -1, axis=0), jnp.roll(x, 1, axis=0)))
    merge64(False)
    return V, I


def _mxu_t(x, m):
    eye = (jax.lax.broadcasted_iota(jnp.int32, (m, m), 0)
           == jax.lax.broadcasted_iota(jnp.int32, (m, m), 1)).astype(jnp.float32)
    return jax.lax.dot_general(x, eye, (((0,), (0,)), ((), ())),
                               precision=jax.lax.Precision.HIGHEST,
                               preferred_element_type=jnp.float32)


def _topk_kernel(cost_ref, vals_ref, eidx_ref):
    n = cost_ref.shape[1]
    blk = pl.program_id(0)
    sub8 = jax.lax.broadcasted_iota(jnp.int32, (8, LANES), 0)

    cT = _mxu_t(cost_ref[...], LANES)
    V = []
    I = []
    for j in range(n // 8):
        V.append(cT[8 * j:8 * j + 8, :])
        I.append(sub8 + (8 * j))
    for j in range(n // 8, NV):
        V.append(jnp.full((8, LANES), jnp.inf, jnp.float32))
        I.append(sub8 + (8 * j))

    V, I = _bitonic_topk(V, I, sub8)

    KP = 56
    av = V[0]
    ai = I[0]
    for u in range(1, 8):
        av = jnp.where(sub8 == u, V[u], av)
        ai = jnp.where(sub8 == u, I[u], ai)
    avs, ais = [av], [ai.astype(jnp.float32)]
    for t in range(1, KP // 8):
        lo = 8 * t
        av = V[lo]
        ai = I[lo]
        for u in range(1, 8):
            j = lo + u
            src_v = V[j] if j < 64 else V[63]
            src_i = I[j] if j < 64 else I[63]
            av = jnp.where(sub8 == u, src_v, av)
            ai = jnp.where(sub8 == u, src_i, ai)
        avs.append(av)
        ais.append(ai.astype(jnp.float32))
    A = jnp.concatenate(avs, axis=0)
    Ai = jnp.concatenate(ais, axis=0)
    outv = _mxu_t(A, KP)[:, :K]
    outi = _mxu_t(Ai, KP)[:, :K].astype(jnp.int32)

    row = blk * LANES + jax.lax.broadcasted_iota(jnp.int32, (LANES, K), 0)
    off = (row // n) * n
    vals_ref[...] = outv
    eidx_ref[0] = row
    eidx_ref[1] = outi + off


def _edge_attr_kernel(v_ref, w_ref, b_ref, out_ref):
    out_ref[...] = v_ref[...] * w_ref[...] + b_ref[...]


def kernel(cost_matrix, init_embedding, W, b):
    B, n, _ = cost_matrix.shape
    R = B * n

    vals, eidx = pl.pallas_call(
        _topk_kernel,
        grid=(R // LANES,),
        in_specs=[pl.BlockSpec((LANES, n), lambda i: (i, 0))],
        out_specs=[
            pl.BlockSpec((LANES, K), lambda i: (i, 0)),
            pl.BlockSpec((2, LANES, K), lambda i: (0, i, 0)),
        ],
        out_shape=[
            jax.ShapeDtypeStruct((R, K), jnp.float32),
            jax.ShapeDtypeStruct((2, R, K), jnp.int32),
        ],
    )(cost_matrix.reshape(R, n))

    E = R * K
    EB = math.gcd(E, 8000)
    attr = pl.pallas_call(
        _edge_attr_kernel,
        grid=(E // EB,),
        in_specs=[
            pl.BlockSpec((EB, 1), lambda i: (i, 0)),
            pl.BlockSpec((1, 64), lambda i: (0, 0)),
            pl.BlockSpec((1, 64), lambda i: (0, 0)),
        ],
        out_specs=pl.BlockSpec((EB, 64), lambda i: (i, 0)),
        out_shape=jax.ShapeDtypeStruct((E, 64), jnp.float32),
    )(vals.reshape(E, 1), W.reshape(1, 64), b.reshape(1, 64))

    x = init_embedding.reshape(R, -1)
    edge_index = eidx.reshape(2, E)
    return x, edge_index, attr


# speedup vs baseline: 4.2808x; 1.3375x over previous
import math

import jax
import jax.numpy as jnp
from jax.experimental import pallas as pl

K = 50
LANES = 128
NV = 128


def _lex_lt(av, ai, bv, bi):
    return (av < bv) | ((av == bv) & (ai < bi))


def _bitonic_topk(V, I, sub8):
    def cx(i, j, desc):
        av, ai, bv, bi = V[i], I[i], V[j], I[j]
        swap = _lex_lt(bv, bi, av, ai)
        if desc is True:
            swap = ~swap
        elif desc is not False:
            swap = swap != desc
        V[i] = jnp.where(swap, bv, av)
        V[j] = jnp.where(swap, av, bv)
        I[i] = jnp.where(swap, bi, ai)
        I[j] = jnp.where(swap, ai, bi)

    for k in (2, 4, 8, 16, 32, 64):
        s = k // 2
        while s >= 1:
            for half, hdesc in ((0, False), (64, True)):
                for q in range(64):
                    if q & s:
                        continue
                    cx(half + q, half + (q | s), ((q & k) != 0) != hdesc)
            s //= 2

    for q in range(64):
        av, ai, bv, bi = V[q], I[q], V[64 + q], I[64 + q]
        bl = _lex_lt(bv, bi, av, ai)
        V[q] = jnp.where(bl, bv, av)
        I[q] = jnp.where(bl, bi, ai)
    V = V[:64]
    I = I[:64]

    def merge64(desc):
        for s in (32, 16, 8, 4, 2, 1):
            for q in range(64):
                if q & s:
                    continue
                cx(q, q | s, desc)

    def prune_sublane(partner_fn):
        for q in range(64):
            pv = partner_fn(V[q])
            pi = partner_fn(I[q])
            bl = _lex_lt(pv, pi, V[q], I[q])
            V[q] = jnp.where(bl, pv, V[q])
            I[q] = jnp.where(bl, pi, I[q])

    merge64((sub8 & 4) != 0)
    prune_sublane(lambda x: jnp.roll(x, 4, axis=0))
    merge64((sub8 & 2) != 0)
    up2 = (sub8 & 2) == 0
    prune_sublane(
        lambda x: jnp.where(up2, jnp.roll(x, -2, axis=0), jnp.roll(x, 2, axis=0)))
    merge64((sub8 & 1) != 0)
    up1 = (sub8 & 1) == 0
    prune_sublane(
        lambda x: jnp.where(up1, jnp.roll(x, -1, axis=0), jnp.roll(x, 1, axis=0)))
    merge64(False)
    return V, I


def _mxu_t(x, m):
    eye = (jax.lax.broadcasted_iota(jnp.int32, (m, m), 0)
           == jax.lax.broadcasted_iota(jnp.int32, (m, m), 1)).astype(jnp.float32)
    return jax.lax.dot_general(x, eye, (((0,), (0,)), ((), ())),
                               precision=jax.lax.Precision.HIGHEST,
                               preferred_element_type=jnp.float32)


def _topk_kernel(cost_ref, w_ref, b_ref, attr_ref, eidx_ref):
    n = cost_ref.shape[1]
    blk = pl.program_id(0)
    sub8 = jax.lax.broadcasted_iota(jnp.int32, (8, LANES), 0)

    cT = _mxu_t(cost_ref[...], LANES)
    V = []
    I = []
    for j in range(n // 8):
        V.append(cT[8 * j:8 * j + 8, :])
        I.append(sub8 + (8 * j))
    for j in range(n // 8, NV):
        V.append(jnp.full((8, LANES), jnp.inf, jnp.float32))
        I.append(sub8 + (8 * j))

    V, I = _bitonic_topk(V, I, sub8)

    KP = 56
    av = V[0]
    ai = I[0]
    for u in range(1, 8):
        av = jnp.where(sub8 == u, V[u], av)
        ai = jnp.where(sub8 == u, I[u], ai)
    avs, ais = [av], [ai.astype(jnp.float32)]
    for t in range(1, KP // 8):
        lo = 8 * t
        av = V[lo]
        ai = I[lo]
        for u in range(1, 8):
            j = lo + u
            src_v = V[j] if j < 64 else V[63]
            src_i = I[j] if j < 64 else I[63]
            av = jnp.where(sub8 == u, src_v, av)
            ai = jnp.where(sub8 == u, src_i, ai)
        avs.append(av)
        ais.append(ai.astype(jnp.float32))
    A = jnp.concatenate(avs, axis=0)
    Ai = jnp.concatenate(ais, axis=0)
    outi = _mxu_t(Ai, KP)[:, :K].astype(jnp.int32)

    row = blk * LANES + jax.lax.broadcasted_iota(jnp.int32, (LANES, K), 0)
    off = (row // n) * n
    eidx_ref[0] = row
    eidx_ref[1] = outi + off

    w = w_ref[...]
    bb = b_ref[...]
    for r in range(LANES):
        attr_ref[K * r:K * (r + 1), :] = A[:K, r:r + 1] * w + bb


def kernel(cost_matrix, init_embedding, W, b):
    B, n, _ = cost_matrix.shape
    R = B * n

    E = R * K
    attr, eidx = pl.pallas_call(
        _topk_kernel,
        grid=(R // LANES,),
        in_specs=[
            pl.BlockSpec((LANES, n), lambda i: (i, 0)),
            pl.BlockSpec((1, 64), lambda i: (0, 0)),
            pl.BlockSpec((1, 64), lambda i: (0, 0)),
        ],
        out_specs=[
            pl.BlockSpec((LANES * K, 64), lambda i: (i, 0)),
            pl.BlockSpec((2, LANES, K), lambda i: (0, i, 0)),
        ],
        out_shape=[
            jax.ShapeDtypeStruct((E, 64), jnp.float32),
            jax.ShapeDtypeStruct((2, R, K), jnp.int32),
        ],
    )(cost_matrix.reshape(R, n), W.reshape(1, 64), b.reshape(1, 64))

    x = init_embedding.reshape(R, -1)
    edge_index = eidx.reshape(2, E)
    return x, edge_index, attr


# speedup vs baseline: 4.2829x; 1.0005x over previous
import jax
import jax.numpy as jnp
from jax.experimental import pallas as pl

K = 50
LANES = 128
NV = 128


def _lex_lt(av, ai, bv, bi):
    return (av < bv) | ((av == bv) & (ai < bi))


def _bitonic_topk(V, I, sub8):
    def cx(i, j, desc):
        av, ai, bv, bi = V[i], I[i], V[j], I[j]
        swap = _lex_lt(bv, bi, av, ai)
        if desc is True:
            swap = ~swap
        elif desc is not False:
            swap = swap != desc
        V[i] = jnp.where(swap, bv, av)
        V[j] = jnp.where(swap, av, bv)
        I[i] = jnp.where(swap, bi, ai)
        I[j] = jnp.where(swap, ai, bi)

    for k in (2, 4, 8, 16, 32, 64):
        s = k // 2
        while s >= 1:
            for half, hdesc in ((0, False), (64, True)):
                for q in range(64):
                    if q & s:
                        continue
                    cx(half + q, half + (q | s), ((q & k) != 0) != hdesc)
            s //= 2

    for q in range(64):
        av, ai, bv, bi = V[q], I[q], V[64 + q], I[64 + q]
        bl = _lex_lt(bv, bi, av, ai)
        V[q] = jnp.where(bl, bv, av)
        I[q] = jnp.where(bl, bi, ai)
    V = V[:64]
    I = I[:64]

    def merge64(desc):
        for s in (32, 16, 8, 4, 2, 1):
            for q in range(64):
                if q & s:
                    continue
                cx(q, q | s, desc)

    def prune_sublane(partner_fn):
        for q in range(64):
            pv = partner_fn(V[q])
            pi = partner_fn(I[q])
            bl = _lex_lt(pv, pi, V[q], I[q])
            V[q] = jnp.where(bl, pv, V[q])
            I[q] = jnp.where(bl, pi, I[q])

    merge64((sub8 & 4) != 0)
    prune_sublane(lambda x: jnp.roll(x, 4, axis=0))
    merge64((sub8 & 2) != 0)
    up2 = (sub8 & 2) == 0
    prune_sublane(
        lambda x: jnp.where(up2, jnp.roll(x, -2, axis=0), jnp.roll(x, 2, axis=0)))
    merge64((sub8 & 1) != 0)
    up1 = (sub8 & 1) == 0
    prune_sublane(
        lambda x: jnp.where(up1, jnp.roll(x, -1, axis=0), jnp.roll(x, 1, axis=0)))
    merge64(False)
    return V, I


def _mxu_t(x, m):
    eye = (jax.lax.broadcasted_iota(jnp.int32, (m, m), 0)
           == jax.lax.broadcasted_iota(jnp.int32, (m, m), 1)).astype(jnp.float32)
    return jax.lax.dot_general(x, eye, (((0,), (0,)), ((), ())),
                               precision=jax.lax.Precision.HIGHEST,
                               preferred_element_type=jnp.float32)


def _topk_kernel(cost_ref, w_ref, b_ref, attr_ref, eidx_ref):
    n = cost_ref.shape[1]
    blk = pl.program_id(0)
    sub8 = jax.lax.broadcasted_iota(jnp.int32, (8, LANES), 0)

    cT = _mxu_t(cost_ref[...], LANES)
    V = []
    I = []
    for j in range(n // 8):
        V.append(cT[8 * j:8 * j + 8, :])
        I.append(sub8 + (8 * j))
    for j in range(n // 8, NV):
        V.append(jnp.full((8, LANES), jnp.inf, jnp.float32))
        I.append(sub8 + (8 * j))

    V, I = _bitonic_topk(V, I, sub8)

    KP = 56
    av = V[0]
    ai = I[0]
    for u in range(1, 8):
        av = jnp.where(sub8 == u, V[u], av)
        ai = jnp.where(sub8 == u, I[u], ai)
    avs, ais = [av], [ai.astype(jnp.float32)]
    for t in range(1, KP // 8):
        lo = 8 * t
        av = V[lo]
        ai = I[lo]
        for u in range(1, 8):
            j = lo + u
            src_v = V[j] if j < 64 else V[63]
            src_i = I[j] if j < 64 else I[63]
            av = jnp.where(sub8 == u, src_v, av)
            ai = jnp.where(sub8 == u, src_i, ai)
        avs.append(av)
        ais.append(ai.astype(jnp.float32))
    A = jnp.concatenate(avs, axis=0)
    Ai = jnp.concatenate(ais, axis=0)
    outi = _mxu_t(Ai, KP)[:, :K].astype(jnp.int32)

    row = blk * LANES + jax.lax.broadcasted_iota(jnp.int32, (LANES, K), 0)
    off = (row // n) * n
    eidx_ref[0] = row
    eidx_ref[1] = outi + off

    w = w_ref[...]
    bb = b_ref[...]
    for r in range(LANES):
        attr_ref[K * r:K * (r + 1), :] = A[:K, r:r + 1] * w + bb


def kernel(cost_matrix, init_embedding, W, b):
    B, n, _ = cost_matrix.shape
    R = B * n

    E = R * K
    attr, eidx = pl.pallas_call(
        _topk_kernel,
        grid=(R // LANES,),
        in_specs=[
            pl.BlockSpec((LANES, n), lambda i: (i, 0)),
            pl.BlockSpec((1, 64), lambda i: (0, 0)),
            pl.BlockSpec((1, 64), lambda i: (0, 0)),
        ],
        out_specs=[
            pl.BlockSpec((LANES * K, 64), lambda i: (i, 0)),
            pl.BlockSpec((2, LANES, K), lambda i: (0, i, 0)),
        ],
        out_shape=[
            jax.ShapeDtypeStruct((E, 64), jnp.float32),
            jax.ShapeDtypeStruct((2, R, K), jnp.int32),
        ],
    )(cost_matrix.reshape(R, n), W.reshape(1, 64), b.reshape(1, 64))

    x = init_embedding.reshape(R, -1)
    edge_index = eidx.reshape(2, E)
    return x, edge_index, attr
